# baseline (speedup 1.0000x reference)
import functools

import jax
import jax.numpy as jnp
from jax import lax
from jax.experimental import pallas as pl
from jax.experimental.pallas import tpu as pltpu
from jax.experimental.pallas import tpu_sc as plsc

_B = 16384
_D = 32
_NBP = 1024
_OW = 2 * _D + 1
_NC, _NS, _L = 2, 16, 16
_NW = _NC * _NS
_RPW = _B // _NW
_CH = 128
_NCH = _RPW // _CH
_STEPS = (512, 256, 128, 64, 32, 16, 8, 4, 2, 1)

_V1 = 1000001
_DT_C = 65536
_DT_R = _DT_C // 4
_DT_STEPS = (_V1 + _DT_C - 1) // _DT_C
_PACKED_ROWS = _DT_STEPS * _DT_R
_DT_C_LOG2 = _DT_C.bit_length() - 1
_DT_R_LOG2 = _DT_R.bit_length() - 1


def _detile_body(x_ref, o_ref):
    x = x_ref[...]
    x2 = jnp.concatenate(
        [x[:, a * _DT_R:(a + 1) * _DT_R] for a in range(4)], axis=0)
    eye = (jax.lax.broadcasted_iota(jnp.int32, (4 * _D, 4 * _D), 0) ==
           jax.lax.broadcasted_iota(jnp.int32, (4 * _D, 4 * _D), 1)
           ).astype(jnp.float32)
    o_ref[...] = jax.lax.dot_general(
        x2, eye, (((0,), (0,)), ((), ())),
        preferred_element_type=jnp.float32)


def _detile(utab_t):
    return pl.pallas_call(
        _detile_body,
        grid=(_DT_STEPS,),
        in_specs=[pl.BlockSpec((_D, _DT_C), lambda i: (0, i))],
        out_specs=pl.BlockSpec((_DT_R, 4 * _D), lambda i: (i, 0)),
        out_shape=jax.ShapeDtypeStruct((_PACKED_ROWS, 4 * _D), jnp.float32),
    )(utab_t)


def _body(uid_hbm, ts_hbm, utab_hbm, ttab_hbm, bkt_hbm, consts_hbm,
          out_hbm,
          ridx_v, offs_v, bidx_v, ts_v, bkt_v, consts_v,
          urows_v, trows_v, out_v, sem_u, sem_t, sem_o):
    wid = lax.axis_index("s") * _NC + lax.axis_index("c")
    base = wid * _RPW

    pltpu.sync_copy(bkt_hbm, bkt_v)
    pltpu.sync_copy(consts_hbm, consts_v)
    pltpu.sync_copy(ts_hbm.at[pl.ds(base, _RPW)], ts_v)
    for j in range(_NCH):
        pltpu.sync_copy(uid_hbm.at[pl.ds(base + j * _CH, _CH)],
                        ridx_v.at[j])

    def fix16(i, _):
        j, o = divmod(i * _L, _CH)
        u = ridx_v[j, pl.ds(o, _L)]
        r = (jnp.right_shift(u, _DT_C_LOG2) * _DT_R
             + jnp.bitwise_and(u, _DT_R - 1))
        off = jnp.bitwise_and(jnp.right_shift(u, _DT_R_LOG2), 3) * _D
        ridx_v[j, pl.ds(o, _L)] = r
        offs_v[pl.ds(i * _L, _L)] = off
        return _

    lax.fori_loop(0, _RPW // _L, fix16, 0, unroll=False)

    ucopies = [
        pltpu.async_copy(utab_hbm.at[ridx_v.at[j]],
                         urows_v.at[pl.ds(j * _CH, _CH)], sem_u)
        for j in range(_NCH)
    ]

    def search16(i, _):
        x = ts_v[pl.ds(i * _L, _L)]
        res = jnp.zeros((_L,), jnp.int32)
        for step in _STEPS:
            nxt = res + step
            b = plsc.load_gather(bkt_v, [nxt - 1])
            res = jnp.where(b <= x, nxt, res)
        bidx_v[i // (_CH // _L), pl.ds((i % (_CH // _L)) * _L, _L)] = res
        return _

    tcopies = []
    for j in range(_NCH):
        lax.fori_loop(j * (_CH // _L), (j + 1) * (_CH // _L), search16, 0,
                      unroll=False)
        tcopies.append(
            pltpu.async_copy(ttab_hbm.at[bidx_v.at[j]],
                             trows_v.at[pl.ds(j * _CH, _CH)], sem_t))

    mean = consts_v[pl.ds(0, _L)]
    std = consts_v[pl.ds(_L, _L)]
    lanes = lax.iota(jnp.int32, _L)

    def asm16(i, _):
        r0 = i * _L
        x = ts_v[pl.ds(r0, _L)]
        v = (x - mean) / std
        rows = r0 + lanes
        plsc.store_scatter(out_v, [rows, jnp.full((_L,), _OW - 1, jnp.int32)],
                           v)
        cols = offs_v[pl.ds(r0, _L)]
        for c in range(_D):
            vals = plsc.load_gather(urows_v, [rows, cols + c])
            plsc.store_scatter(out_v, [rows, jnp.full((_L,), c, jnp.int32)],
                               vals)
        for rl in range(_L):
            r = r0 + rl
            out_v[r, pl.ds(2 * _L, _L)] = trows_v[r, pl.ds(0, _L)]
            out_v[r, pl.ds(3 * _L, _L)] = trows_v[r, pl.ds(_L, _L)]
        return _

    ocopies = []
    for j in range(_NCH):
        ucopies[j].wait()
        tcopies[j].wait()
        lax.fori_loop(j * (_CH // _L), (j + 1) * (_CH // _L), asm16, 0,
                      unroll=False)
        ocopies.append(
            pltpu.async_copy(out_v.at[pl.ds(j * _CH, _CH)],
                             out_hbm.at[pl.ds(base + j * _CH, _CH)], sem_o))
    for c in ocopies:
        c.wait()


@jax.jit
def _sc_call(uid, ts, utab, ttab, bkt_pad, consts):
    mesh = plsc.VectorSubcoreMesh(core_axis_name="c", subcore_axis_name="s")
    f = pl.kernel(
        _body,
        out_type=jax.ShapeDtypeStruct((_B, _OW), jnp.float32),
        mesh=mesh,
        compiler_params=pltpu.CompilerParams(needs_layout_passes=False,
                                             use_tc_tiling_on_sc=False),
        scratch_types=[
            pltpu.VMEM((_NCH, _CH), jnp.int32),
            pltpu.VMEM((_RPW,), jnp.int32),
            pltpu.VMEM((_NCH, _CH), jnp.int32),
            pltpu.VMEM((_RPW,), jnp.float32),
            pltpu.VMEM((_NBP,), jnp.float32),
            pltpu.VMEM((2 * _L,), jnp.float32),
            pltpu.VMEM((_RPW, 4 * _D), jnp.float32),
            pltpu.VMEM((_RPW, _D), jnp.float32),
            pltpu.VMEM((_RPW, _OW), jnp.float32),
            pltpu.SemaphoreType.DMA,
            pltpu.SemaphoreType.DMA,
            pltpu.SemaphoreType.DMA,
        ],
    )
    return f(uid, ts, utab, ttab, bkt_pad, consts)


def kernel(user_id, time_stamp, user_table, ts_table, buckets, ts_mean, ts_std):
    uid = user_id.astype(jnp.int32)
    nb = buckets.shape[0]
    bkt_pad = jnp.concatenate(
        [buckets.astype(jnp.float32),
         jnp.full((_NBP - nb,), jnp.inf, jnp.float32)])
    consts = jnp.concatenate(
        [jnp.full((_L,), ts_mean, jnp.float32),
         jnp.full((_L,), ts_std, jnp.float32)])
    return _sc_call(uid, time_stamp.astype(jnp.float32),
                    _detile(user_table.T), ts_table, bkt_pad, consts)


# speedup vs baseline: 1.0008x; 1.0008x over previous
import jax
import jax.numpy as jnp
from jax import lax
from jax.experimental import pallas as pl
from jax.experimental.pallas import tpu as pltpu
from jax.experimental.pallas import tpu_sc as plsc

_B = 16384
_D = 32
_NBP = 1024
_OW = 2 * _D + 1
_NC, _NS, _L = 2, 16, 16
_NW = _NC * _NS
_RPW = _B // _NW
_CH = 128
_NCH = _RPW // _CH
_STEPS = (512, 256, 128, 64, 32, 16, 8, 4, 2, 1)

_V1 = 1000001
_DT_C = 65536
_DT_R = _DT_C // 4
_DT_STEPS = (_V1 + _DT_C - 1) // _DT_C
_PACKED_ROWS = _DT_STEPS * _DT_R
_DT_C_LOG2 = _DT_C.bit_length() - 1
_DT_R_LOG2 = _DT_R.bit_length() - 1


def _detile_body(x_ref, o_ref):
    x = x_ref[...]
    x2 = jnp.concatenate(
        [x[:, a * _DT_R:(a + 1) * _DT_R] for a in range(4)], axis=0)
    eye = (jax.lax.broadcasted_iota(jnp.int32, (4 * _D, 4 * _D), 0) ==
           jax.lax.broadcasted_iota(jnp.int32, (4 * _D, 4 * _D), 1)
           ).astype(jnp.float32)
    o_ref[...] = jax.lax.dot_general(
        x2, eye, (((0,), (0,)), ((), ())),
        preferred_element_type=jnp.float32)


def _detile(utab_t):
    return pl.pallas_call(
        _detile_body,
        grid=(_DT_STEPS,),
        in_specs=[pl.BlockSpec((_D, _DT_C), lambda i: (0, i))],
        out_specs=pl.BlockSpec((_DT_R, 4 * _D), lambda i: (i, 0)),
        out_shape=jax.ShapeDtypeStruct((_PACKED_ROWS, 4 * _D), jnp.float32),
    )(utab_t)


def _body(uid_hbm, ts_hbm, utab_hbm, ttab_hbm, bkt_hbm, consts_hbm,
          out_hbm,
          ridx_v, offs_v, bidx_v, ts_v, bkt_v, consts_v,
          urows_v, trows_v, out_v, sem_u, sem_t, sem_o):
    wid = lax.axis_index("s") * _NC + lax.axis_index("c")
    base = wid * _RPW

    pltpu.sync_copy(bkt_hbm, bkt_v)
    pltpu.sync_copy(consts_hbm, consts_v)
    pltpu.sync_copy(ts_hbm.at[pl.ds(base, _RPW)], ts_v)
    for j in range(_NCH):
        pltpu.sync_copy(uid_hbm.at[pl.ds(base + j * _CH, _CH)],
                        ridx_v.at[j])

    def fix16(i, _):
        j, o = divmod(i * _L, _CH)
        u = ridx_v[j, pl.ds(o, _L)]
        r = (jnp.right_shift(u, _DT_C_LOG2) * _DT_R
             + jnp.bitwise_and(u, _DT_R - 1))
        off = jnp.bitwise_and(jnp.right_shift(u, _DT_R_LOG2), 3) * _D
        ridx_v[j, pl.ds(o, _L)] = r
        offs_v[pl.ds(i * _L, _L)] = off
        return _

    lax.fori_loop(0, _RPW // _L, fix16, 0, unroll=False)

    ucopies = [
        pltpu.async_copy(utab_hbm.at[ridx_v.at[j]],
                         urows_v.at[pl.ds(j * _CH, _CH)], sem_u)
        for j in range(_NCH)
    ]

    def search16(i, _):
        x = ts_v[pl.ds(i * _L, _L)]
        res = jnp.zeros((_L,), jnp.int32)
        for step in _STEPS:
            nxt = res + step
            b = plsc.load_gather(bkt_v, [nxt - 1])
            res = jnp.where(b <= x, nxt, res)
        bidx_v[i // (_CH // _L), pl.ds((i % (_CH // _L)) * _L, _L)] = res
        return _

    tcopies = []
    for j in range(_NCH):
        lax.fori_loop(j * (_CH // _L), (j + 1) * (_CH // _L), search16, 0,
                      unroll=False)
        tcopies.append(
            pltpu.async_copy(ttab_hbm.at[bidx_v.at[j]],
                             trows_v.at[pl.ds(j * _CH, _CH)], sem_t))

    mean = consts_v[pl.ds(0, _L)]
    std = consts_v[pl.ds(_L, _L)]
    lanes = lax.iota(jnp.int32, _L)

    def asm16(i, _):
        r0 = i * _L
        x = ts_v[pl.ds(r0, _L)]
        v = (x - mean) / std
        rows = r0 + lanes
        plsc.store_scatter(out_v, [rows, jnp.full((_L,), _OW - 1, jnp.int32)],
                           v)
        cols = offs_v[pl.ds(r0, _L)]
        for c in range(_D):
            vals = plsc.load_gather(urows_v, [rows, cols + c])
            plsc.store_scatter(out_v, [rows, jnp.full((_L,), c, jnp.int32)],
                               vals)
        for rl in range(_L):
            r = r0 + rl
            out_v[r, pl.ds(2 * _L, _L)] = trows_v[r, pl.ds(0, _L)]
            out_v[r, pl.ds(3 * _L, _L)] = trows_v[r, pl.ds(_L, _L)]
        return _

    ocopies = []
    for j in range(_NCH):
        ucopies[j].wait()
        tcopies[j].wait()
        lax.fori_loop(j * (_CH // _L), (j + 1) * (_CH // _L), asm16, 0,
                      unroll=False)
        ocopies.append(
            pltpu.async_copy(out_v.at[pl.ds(j * _CH, _CH)],
                             out_hbm.at[pl.ds(base + j * _CH, _CH)], sem_o))
    for c in ocopies:
        c.wait()


@jax.jit
def _sc_call(uid, ts, utab, ttab, bkt_pad, consts):
    mesh = plsc.VectorSubcoreMesh(core_axis_name="c", subcore_axis_name="s")
    f = pl.kernel(
        _body,
        out_type=jax.ShapeDtypeStruct((_B, _OW), jnp.float32),
        mesh=mesh,
        compiler_params=pltpu.CompilerParams(needs_layout_passes=False,
                                             use_tc_tiling_on_sc=False),
        scratch_types=[
            pltpu.VMEM((_NCH, _CH), jnp.int32),
            pltpu.VMEM((_RPW,), jnp.int32),
            pltpu.VMEM((_NCH, _CH), jnp.int32),
            pltpu.VMEM((_RPW,), jnp.float32),
            pltpu.VMEM((_NBP,), jnp.float32),
            pltpu.VMEM((2 * _L,), jnp.float32),
            pltpu.VMEM((_RPW, 4 * _D), jnp.float32),
            pltpu.VMEM((_RPW, _D), jnp.float32),
            pltpu.VMEM((_RPW, _OW), jnp.float32),
            pltpu.SemaphoreType.DMA,
            pltpu.SemaphoreType.DMA,
            pltpu.SemaphoreType.DMA,
        ],
    )
    return f(uid, ts, utab, ttab, bkt_pad, consts)


def kernel(user_id, time_stamp, user_table, ts_table, buckets, ts_mean, ts_std):
    uid = user_id.astype(jnp.int32)
    nb = buckets.shape[0]
    bkt_pad = jnp.concatenate(
        [buckets.astype(jnp.float32),
         jnp.full((_NBP - nb,), jnp.inf, jnp.float32)])
    consts = jnp.concatenate(
        [jnp.full((_L,), ts_mean, jnp.float32),
         jnp.full((_L,), ts_std, jnp.float32)])
    return _sc_call(uid, time_stamp.astype(jnp.float32),
                    _detile(user_table.T), ts_table, bkt_pad, consts)
